# baseline (speedup 1.0000x reference)
import functools

import jax
import jax.numpy as jnp
from jax import lax
from jax.experimental import pallas as pl
from jax.experimental.pallas import tpu as pltpu
from jax.experimental.pallas import tpu_sc as plsc

VOCAB = 1000
BATCH = 4096
HIST = 20
NC, NS = 2, 16
NW = NC * NS
SUPER = 2
BW = 128 * SUPER
NSUP = BATCH // BW
TSPAN = HIST * NSUP // NW
VC = 200
VCHUNKS = VOCAB // VC
NCHUNK = TSPAN * VCHUNKS
NBUF = 2

_mesh = plsc.VectorSubcoreMesh(core_axis_name="c", subcore_axis_name="s")


@functools.partial(
    pl.kernel,
    mesh=_mesh,
    out_type=jax.ShapeDtypeStruct((HIST, VOCAB, BATCH), jnp.float32),
    scratch_types=[
        pltpu.VMEM((TSPAN, BW), jnp.int32),
        [pltpu.VMEM((VC, BW), jnp.float32)] * NBUF,
        [pltpu.SemaphoreType.DMA] * NBUF,
    ],
    compiler_params=pltpu.CompilerParams(needs_layout_passes=False),
)
def _onehot_sc(arrt_hbm, out_hbm, tcol, bufs, sems):
    wid = lax.axis_index("s") * NC + lax.axis_index("c")
    sup = wid % NSUP
    t0 = (wid // NSUP) * TSPAN
    b0 = sup * BW

    pltpu.sync_copy(arrt_hbm.at[pl.ds(t0, TSPAN), sup], tcol)

    zeros16 = jnp.zeros((16,), jnp.float32)

    def _zfill(j, carry):
        r = j * 16 // BW
        c = j * 16 % BW
        for b in range(NBUF):
            bufs[b][r, pl.ds(c, 16)] = zeros16
        return carry

    lax.fori_loop(0, VC * BW // 16, _zfill, 0)

    iota16 = lax.iota(jnp.int32, 16)
    ones16 = jnp.ones((16,), jnp.float32)

    def _scatter(buf, g, val):
        t = g // VCHUNKS
        v0 = (g % VCHUNKS) * VC
        for i in range(BW // 16):
            cols = tcol[t, pl.ds(i * 16, 16)]
            m = (cols >= v0) & (cols < v0 + VC)
            vloc = jnp.where(m, cols - v0, 0)
            plsc.store_scatter(buf, [vloc, iota16 + i * 16], val, mask=m)

    def _start_out(b, g):
        t = g // VCHUNKS
        v0 = (g % VCHUNKS) * VC
        pltpu.make_async_copy(
            bufs[b], out_hbm.at[t0 + t, pl.ds(v0, VC), pl.ds(b0, BW)], sems[b]
        ).start()

    def _wait_out(b):
        pltpu.make_async_copy(
            bufs[b], out_hbm.at[0, pl.ds(0, VC), pl.ds(0, BW)], sems[b]
        ).wait()

    for b in range(NBUF):
        _scatter(bufs[b], b, ones16)
        _start_out(b, b)

    def _chunk_body(k, carry):
        g0 = NBUF + k * NBUF
        for b in range(NBUF):
            g = g0 + b
            _wait_out(b)
            _scatter(bufs[b], g - NBUF, zeros16)
            _scatter(bufs[b], g, ones16)
            _start_out(b, g)
        return carry

    lax.fori_loop(0, (NCHUNK - NBUF) // NBUF, _chunk_body, 0)

    for b in range(NBUF):
        _wait_out(b)


def kernel(arr, mask):
    del mask
    arrt = jnp.transpose(arr.astype(jnp.int32), (1, 0)).reshape(HIST, NSUP, BW)
    out3 = _onehot_sc(arrt)
    return jnp.transpose(out3, (2, 0, 1))


# speedup vs baseline: 1.0501x; 1.0501x over previous
import functools

import jax
import jax.numpy as jnp
from jax import lax
from jax.experimental import pallas as pl
from jax.experimental.pallas import tpu as pltpu
from jax.experimental.pallas import tpu_sc as plsc

VOCAB = 1000
BATCH = 4096
HIST = 20
NC, NS = 2, 16
NW = NC * NS
BW = BATCH // NW
VC = 200
VCHUNKS = VOCAB // VC
NCHUNK = HIST * VCHUNKS
NBUF = 5

_mesh = plsc.VectorSubcoreMesh(core_axis_name="c", subcore_axis_name="s")


@functools.partial(
    pl.kernel,
    mesh=_mesh,
    out_type=jax.ShapeDtypeStruct((HIST, VOCAB, BATCH), jnp.float32),
    scratch_types=[
        pltpu.VMEM((HIST, BW), jnp.int32),
        [pltpu.VMEM((VC, BW), jnp.float32)] * NBUF,
        [pltpu.SemaphoreType.DMA] * NBUF,
    ],
    compiler_params=pltpu.CompilerParams(needs_layout_passes=False),
)
def _onehot_sc(arrt_hbm, out_hbm, tcol, bufs, sems):
    wid = lax.axis_index("s") * NC + lax.axis_index("c")
    b0 = wid * BW

    pltpu.sync_copy(arrt_hbm.at[:, wid], tcol)

    zeros16 = jnp.zeros((16,), jnp.float32)

    def _zfill(j, carry):
        r = j * 16 // BW
        c = j * 16 % BW
        for b in range(NBUF):
            bufs[b][r, pl.ds(c, 16)] = zeros16
        return carry

    lax.fori_loop(0, VC * BW // 16, _zfill, 0)

    iota16 = lax.iota(jnp.int32, 16)
    ones16 = jnp.ones((16,), jnp.float32)

    def _scatter(buf, g, val):
        t = g // VCHUNKS
        v0 = (g % VCHUNKS) * VC
        for i in range(BW // 16):
            cols = tcol[t, pl.ds(i * 16, 16)]
            m = (cols >= v0) & (cols < v0 + VC)
            vloc = jnp.where(m, cols - v0, 0)
            plsc.store_scatter(buf, [vloc, iota16 + i * 16], val, mask=m)

    def _start_out(b, g):
        t = g // VCHUNKS
        v0 = (g % VCHUNKS) * VC
        pltpu.make_async_copy(
            bufs[b], out_hbm.at[t, pl.ds(v0, VC), pl.ds(b0, BW)], sems[b]
        ).start()

    def _wait_out(b):
        pltpu.make_async_copy(
            bufs[b], out_hbm.at[0, pl.ds(0, VC), pl.ds(0, BW)], sems[b]
        ).wait()

    for b in range(NBUF):
        _scatter(bufs[b], b, ones16)
        _start_out(b, b)

    def _chunk_body(k, carry):
        g0 = NBUF + k * NBUF
        for b in range(NBUF):
            g = g0 + b
            _wait_out(b)
            _scatter(bufs[b], g - NBUF, zeros16)
            _scatter(bufs[b], g, ones16)
            _start_out(b, g)
        return carry

    lax.fori_loop(0, (NCHUNK - NBUF) // NBUF, _chunk_body, 0)

    for b in range(NBUF):
        _wait_out(b)


def kernel(arr, mask):
    del mask
    arrt = jnp.transpose(arr.astype(jnp.int32), (1, 0)).reshape(HIST, NW, BW)
    out3 = _onehot_sc(arrt)
    return jnp.transpose(out3, (2, 0, 1))


# speedup vs baseline: 1.1034x; 1.0507x over previous
import functools

import jax
import jax.numpy as jnp
from jax import lax
from jax.experimental import pallas as pl
from jax.experimental.pallas import tpu as pltpu
from jax.experimental.pallas import tpu_sc as plsc

VOCAB = 1000
BATCH = 4096
HIST = 20
NC, NS = 2, 16
NW = NC * NS
BW = BATCH // NW
VC = 200
VCHUNKS = VOCAB // VC
NCHUNK = HIST * VCHUNKS
NBUF = 5

_mesh = plsc.VectorSubcoreMesh(core_axis_name="c", subcore_axis_name="s")


@functools.partial(
    pl.kernel,
    mesh=_mesh,
    out_type=jax.ShapeDtypeStruct((HIST, VOCAB, BATCH), jnp.float32),
    scratch_types=[
        pltpu.VMEM((HIST, BW), jnp.int32),
        [pltpu.VMEM((VC, BW), jnp.float32)] * NBUF,
        [pltpu.SemaphoreType.DMA] * NBUF,
        pltpu.SemaphoreType.DMA,
    ],
    compiler_params=pltpu.CompilerParams(needs_layout_passes=False),
)
def _onehot_sc(arrt_hbm, out_hbm, tcol, bufs, sems, insem):
    wid = lax.axis_index("s") * NC + lax.axis_index("c")
    b0 = wid * BW

    in_copy = pltpu.make_async_copy(
        arrt_hbm.at[:, pl.ds(b0, BW)], tcol, insem
    )
    in_copy.start()

    zeros16 = jnp.zeros((16,), jnp.float32)
    iota16 = lax.iota(jnp.int32, 16)
    ones16 = jnp.ones((16,), jnp.float32)

    def _zfill(buf):
        def body(r, carry):
            for c in range(BW // 16):
                buf[r, pl.ds(c * 16, 16)] = zeros16
            return carry
        lax.fori_loop(0, VC, body, 0)

    def _scatter(buf, g, val):
        t = g // VCHUNKS
        v0 = (g % VCHUNKS) * VC
        for i in range(BW // 16):
            cols = tcol[t, pl.ds(i * 16, 16)]
            m = (cols >= v0) & (cols < v0 + VC)
            vloc = jnp.where(m, cols - v0, 0)
            plsc.store_scatter(buf, [vloc, iota16 + i * 16], val, mask=m)

    def _start_out(b, g):
        t = g // VCHUNKS
        v0 = (g % VCHUNKS) * VC
        pltpu.make_async_copy(
            bufs[b], out_hbm.at[t, pl.ds(v0, VC), pl.ds(b0, BW)], sems[b]
        ).start()

    def _wait_out(b):
        pltpu.make_async_copy(
            bufs[b], out_hbm.at[0, pl.ds(0, VC), pl.ds(0, BW)], sems[b]
        ).wait()

    _zfill(bufs[0])
    in_copy.wait()
    _scatter(bufs[0], 0, ones16)
    _start_out(0, 0)
    for b in range(1, NBUF):
        _zfill(bufs[b])
        _scatter(bufs[b], b, ones16)
        _start_out(b, b)

    def _chunk_body(k, carry):
        g0 = NBUF + k * NBUF
        for b in range(NBUF):
            g = g0 + b
            _wait_out(b)
            _scatter(bufs[b], g - NBUF, zeros16)
            _scatter(bufs[b], g, ones16)
            _start_out(b, g)
        return carry

    lax.fori_loop(0, (NCHUNK - NBUF) // NBUF, _chunk_body, 0)

    for b in range(NBUF):
        _wait_out(b)


def kernel(arr, mask):
    del mask
    arrt = jnp.transpose(arr.astype(jnp.int32), (1, 0))
    out3 = _onehot_sc(arrt)
    return jnp.transpose(out3, (2, 0, 1))
